# baseline (speedup 1.0000x reference)
import functools

import jax
import jax.numpy as jnp
import numpy as np
from jax import lax
from jax.experimental import pallas as pl
from jax.experimental.pallas import tpu as pltpu
from jax.experimental.pallas import tpu_sc as plsc

N = 10000
E = 320000
D = 128
NC = 2
NS = 16
NW = NC * NS
CHUNK = 128
CPT = 80
RING = 8
E_PAD = NW * CPT * CHUNK
N_PAD = 10240
QSCALE = 4096.0
RPT = N_PAD // NS
BM = 1024


def _quantize(y):
    q = jnp.clip(jnp.round(y * QSCALE), -32768.0, 32767.0)
    return q.astype(jnp.int16)


def _mm_body(x_ref, w_ref, o_ref):
    y = jnp.dot(x_ref[...], w_ref[...], preferred_element_type=jnp.float32)
    if o_ref.dtype == jnp.int16:
        o_ref[...] = _quantize(y)
    else:
        o_ref[...] = y.astype(o_ref.dtype)


def _tc_matmul(x, w, out_dtype=jnp.float32):
    m, k = x.shape
    n = w.shape[1]
    return pl.pallas_call(
        _mm_body,
        grid=(m // BM,),
        in_specs=[
            pl.BlockSpec((BM, k), lambda i: (i, 0)),
            pl.BlockSpec((k, n), lambda i: (0, 0)),
        ],
        out_specs=pl.BlockSpec((BM, n), lambda i: (i, 0)),
        out_shape=jax.ShapeDtypeStruct((m, n), out_dtype),
    )(x, w)


def _sc_msgpass(hw, src3, dst3):
    mesh = plsc.VectorSubcoreMesh(core_axis_name="c", subcore_axis_name="s",
                                  num_cores=NC, num_subcores=NS)

    out_type = jax.ShapeDtypeStruct((NC, N_PAD, D), jnp.float32)
    scratch = [
        pltpu.VMEM((CPT, CHUNK), jnp.int32),
        pltpu.VMEM((CPT, CHUNK), jnp.int32),
        pltpu.VMEM((CHUNK, D), jnp.float32),
        pltpu.VMEM_SHARED((N_PAD, D), jnp.float32),
        pltpu.SemaphoreType.DMA,
    ]

    def body(hw_hbm, src_hbm, dst_hbm, out_hbm, src_v, dst_v, rows_v, acc,
             sem):
        cid = lax.axis_index("c")
        sid = lax.axis_index("s")
        wid = cid * NS + sid

        pltpu.sync_copy(src_hbm.at[wid], src_v)
        pltpu.sync_copy(dst_hbm.at[wid], dst_v)

        zero16 = jnp.zeros((16,), jnp.float32)

        def zrow(i, carry):
            for j in range(D // 16):
                rows_v[i, pl.ds(j * 16, 16)] = zero16
            return carry
        lax.fori_loop(0, CHUNK, zrow, 0)

        base = sid * RPT
        for k in range(RPT // CHUNK):
            pltpu.sync_copy(rows_v, acc.at[pl.ds(base + k * CHUNK, CHUNK)])

        plsc.subcore_barrier()

        def step(j, carry):
            pltpu.async_copy(hw_hbm.at[src_v.at[j]], rows_v, sem).wait()
            pltpu.sync_copy(rows_v, acc.at[dst_v.at[j]], add=True)
            return carry
        lax.fori_loop(0, CPT, step, 0)

        plsc.subcore_barrier()

        pltpu.sync_copy(acc.at[pl.ds(base, RPT)],
                        out_hbm.at[cid].at[pl.ds(base, RPT)])

    f = pl.kernel(body, out_type=out_type, mesh=mesh, scratch_types=scratch)
    return f(hw, src3, dst3)


def _sc_degree(dst3):
    mesh = plsc.VectorSubcoreMesh(core_axis_name="c", subcore_axis_name="s",
                                  num_cores=NC, num_subcores=NS)

    def body(dst_hbm, zo_hbm, deg_hbm, dst_v, ones_v, dacc):
        cid = lax.axis_index("c")
        sid = lax.axis_index("s")
        wid = cid * NS + sid

        pltpu.sync_copy(dst_hbm.at[wid], dst_v)
        pltpu.sync_copy(zo_hbm.at[0], ones_v)
        base = sid * RPT
        for k in range(RPT // CHUNK):
            pltpu.sync_copy(ones_v, dacc.at[pl.ds(base + k * CHUNK, CHUNK)])
        pltpu.sync_copy(zo_hbm.at[1], ones_v)

        plsc.subcore_barrier()

        def step(j, carry):
            pltpu.sync_copy(ones_v, dacc.at[dst_v.at[j]], add=True)
            return carry
        lax.fori_loop(0, CPT, step, 0)

        plsc.subcore_barrier()

        pltpu.sync_copy(dacc.at[pl.ds(base, RPT)],
                        deg_hbm.at[cid].at[pl.ds(base, RPT)])

    f = pl.kernel(
        body,
        out_type=jax.ShapeDtypeStruct((NC, N_PAD, 8), jnp.float32),
        mesh=mesh,
        scratch_types=[
            pltpu.VMEM((CPT, CHUNK), jnp.int32),
            pltpu.VMEM((CHUNK, 8), jnp.float32),
            pltpu.VMEM_SHARED((N_PAD, 8), jnp.float32),
        ],
        compiler_params=pltpu.CompilerParams(use_tc_tiling_on_sc=False),
    )
    zo = jnp.concatenate([jnp.zeros((1, CHUNK, 8), jnp.float32),
                          jnp.ones((1, CHUNK, 8), jnp.float32)])
    return f(dst3, zo)


def _comb1_body(agg_ref, degp_ref, w_ref, h1_ref, hw2_ref, deg_ref):
    agg = agg_ref[0] + agg_ref[1]
    deg = jnp.maximum(degp_ref[0, :, :1] + degp_ref[1, :, :1], 1.0)
    h1 = jnp.maximum(agg, 0.0) / deg
    h1_ref[...] = h1
    hw2_ref[...] = jnp.dot(h1, w_ref[...], preferred_element_type=jnp.float32)
    deg_ref[...] = deg


def _tc_combine1(agg_parts, deg_parts, w_enc2):
    return pl.pallas_call(
        _comb1_body,
        grid=(N_PAD // BM,),
        in_specs=[
            pl.BlockSpec((NC, BM, D), lambda i: (0, i, 0)),
            pl.BlockSpec((NC, BM, 8), lambda i: (0, i, 0)),
            pl.BlockSpec((D, D), lambda i: (0, 0)),
        ],
        out_specs=[
            pl.BlockSpec((BM, D), lambda i: (i, 0)),
            pl.BlockSpec((BM, D), lambda i: (i, 0)),
            pl.BlockSpec((BM, 1), lambda i: (i, 0)),
        ],
        out_shape=[
            jax.ShapeDtypeStruct((N_PAD, D), jnp.float32),
            jax.ShapeDtypeStruct((N_PAD, D), jnp.float32),
            jax.ShapeDtypeStruct((N_PAD, 1), jnp.float32),
        ],
    )(agg_parts, deg_parts, w_enc2)


def _dec_body(agg_ref, deg_ref, h1_ref, wda_ref, wdb_ref, wm_ref,
              wc_ref, wv_ref, ws_ref, logits_ref, c_ref, v_ref, f_ref):
    agg = agg_ref[0] + agg_ref[1]
    deg = deg_ref[...]
    h2 = jnp.maximum(agg, 0.0) / deg
    hd = jnp.dot(h2, wda_ref[...], preferred_element_type=jnp.float32)
    hd = hd + jnp.dot(h1_ref[...], wdb_ref[...],
                      preferred_element_type=jnp.float32)
    hd = jnp.maximum(hd, 0.0)
    y = jnp.dot(hd, wm_ref[...], preferred_element_type=jnp.float32)
    f = jnp.where(y >= 0.0, y, 0.1 * y)
    f_ref[...] = f
    cz = jnp.dot(f, wc_ref[...], preferred_element_type=jnp.float32)
    c_ref[...] = 1.0 / (1.0 + jnp.exp(-cz))
    vz = jnp.dot(f, wv_ref[...], preferred_element_type=jnp.float32)
    v_ref[...] = jnp.maximum(vz, 0.0)
    logits_ref[...] = jnp.dot(f, ws_ref[...],
                              preferred_element_type=jnp.float32)


def _tc_decode(agg2_parts, deg, h1, w_dec, w_mlp, w_center, w_var, w_softmax):
    wda = w_dec[:D]
    wdb = w_dec[D:]
    nc1 = w_center.shape[1]
    nv = w_var.shape[1]
    ns = w_softmax.shape[1]
    return pl.pallas_call(
        _dec_body,
        grid=(N_PAD // BM,),
        in_specs=[
            pl.BlockSpec((NC, BM, D), lambda i: (0, i, 0)),
            pl.BlockSpec((BM, 1), lambda i: (i, 0)),
            pl.BlockSpec((BM, D), lambda i: (i, 0)),
            pl.BlockSpec((D, D), lambda i: (0, 0)),
            pl.BlockSpec((D, D), lambda i: (0, 0)),
            pl.BlockSpec((D, D), lambda i: (0, 0)),
            pl.BlockSpec((D, nc1), lambda i: (0, 0)),
            pl.BlockSpec((D, nv), lambda i: (0, 0)),
            pl.BlockSpec((D, ns), lambda i: (0, 0)),
        ],
        out_specs=[
            pl.BlockSpec((BM, ns), lambda i: (i, 0)),
            pl.BlockSpec((BM, nc1), lambda i: (i, 0)),
            pl.BlockSpec((BM, nv), lambda i: (i, 0)),
            pl.BlockSpec((BM, D), lambda i: (i, 0)),
        ],
        out_shape=[
            jax.ShapeDtypeStruct((N_PAD, ns), jnp.float32),
            jax.ShapeDtypeStruct((N_PAD, nc1), jnp.float32),
            jax.ShapeDtypeStruct((N_PAD, nv), jnp.float32),
            jax.ShapeDtypeStruct((N_PAD, D), jnp.float32),
        ],
    )(agg2_parts, deg, h1, wda, wdb, w_mlp, w_center, w_var, w_softmax)


def kernel(x, edge_index, W_enc1, W_enc2, W_dec, W_mlp, W_center, W_var,
           W_softmax):
    src = edge_index[0]
    dst = edge_index[1]
    pad = E_PAD - E
    pad_src = jnp.arange(pad, dtype=jnp.int32) % N
    pad_dst = N + jnp.arange(pad, dtype=jnp.int32) % (N_PAD - N)
    src3 = jnp.concatenate([src, pad_src]).reshape(NW, CPT, CHUNK)
    dst3 = jnp.concatenate([dst, pad_dst]).reshape(NW, CPT, CHUNK)

    x_p = jnp.concatenate([x, jnp.zeros((N_PAD - N, D), jnp.float32)])
    hw1 = _tc_matmul(x_p, W_enc1)
    deg_parts = _sc_degree(dst3)
    agg1_parts = _sc_msgpass(hw1, src3, dst3)
    h1, hw2, deg = _tc_combine1(agg1_parts, deg_parts, W_enc2)
    agg2_parts = _sc_msgpass(hw2, src3, dst3)
    logits, c, v, f = _tc_decode(agg2_parts, deg, h1, W_dec, W_mlp,
                                 W_center, W_var, W_softmax)
    return (logits[:N], c[:N], v[:N], f[:N])


# speedup vs baseline: 1.0103x; 1.0103x over previous
import functools

import jax
import jax.numpy as jnp
import numpy as np
from jax import lax
from jax.experimental import pallas as pl
from jax.experimental.pallas import tpu as pltpu
from jax.experimental.pallas import tpu_sc as plsc

N = 10000
E = 320000
D = 128
NC = 2
NS = 16
NW = NC * NS
CHUNK = 128
CPT = 80
RING = 8
E_PAD = NW * CPT * CHUNK
N_PAD = 10240
QSCALE = 4096.0
RPT = N_PAD // NS
BM = 2048


def _quantize(y):
    q = jnp.clip(jnp.round(y * QSCALE), -32768.0, 32767.0)
    return q.astype(jnp.int16)


def _mm_body(x_ref, w_ref, o_ref):
    y = jnp.dot(x_ref[...], w_ref[...], preferred_element_type=jnp.float32)
    if o_ref.dtype == jnp.int16:
        o_ref[...] = _quantize(y)
    else:
        o_ref[...] = y.astype(o_ref.dtype)


def _tc_matmul(x, w, out_dtype=jnp.float32):
    m, k = x.shape
    n = w.shape[1]
    return pl.pallas_call(
        _mm_body,
        grid=(m // BM,),
        in_specs=[
            pl.BlockSpec((BM, k), lambda i: (i, 0)),
            pl.BlockSpec((k, n), lambda i: (0, 0)),
        ],
        out_specs=pl.BlockSpec((BM, n), lambda i: (i, 0)),
        out_shape=jax.ShapeDtypeStruct((m, n), out_dtype),
    )(x, w)


def _sc_msgpass(hw, src3, dst3):
    mesh = plsc.VectorSubcoreMesh(core_axis_name="c", subcore_axis_name="s",
                                  num_cores=NC, num_subcores=NS)

    out_type = jax.ShapeDtypeStruct((NC, N_PAD, D), jnp.float32)
    scratch = [
        pltpu.VMEM((CPT, CHUNK), jnp.int32),
        pltpu.VMEM((CPT, CHUNK), jnp.int32),
        pltpu.VMEM((CHUNK, D), jnp.float32),
        pltpu.VMEM_SHARED((N_PAD, D), jnp.float32),
        pltpu.SemaphoreType.DMA,
    ]

    def body(hw_hbm, src_hbm, dst_hbm, out_hbm, src_v, dst_v, rows_v, acc,
             sem):
        cid = lax.axis_index("c")
        sid = lax.axis_index("s")
        wid = cid * NS + sid

        pltpu.sync_copy(src_hbm.at[wid], src_v)
        pltpu.sync_copy(dst_hbm.at[wid], dst_v)

        zero16 = jnp.zeros((16,), jnp.float32)

        def zrow(i, carry):
            for j in range(D // 16):
                rows_v[i, pl.ds(j * 16, 16)] = zero16
            return carry
        lax.fori_loop(0, CHUNK, zrow, 0)

        base = sid * RPT
        for k in range(RPT // CHUNK):
            pltpu.sync_copy(rows_v, acc.at[pl.ds(base + k * CHUNK, CHUNK)])

        plsc.subcore_barrier()

        def step(j, carry):
            pltpu.async_copy(hw_hbm.at[src_v.at[j]], rows_v, sem).wait()
            pltpu.sync_copy(rows_v, acc.at[dst_v.at[j]], add=True)
            return carry
        lax.fori_loop(0, CPT, step, 0)

        plsc.subcore_barrier()

        pltpu.sync_copy(acc.at[pl.ds(base, RPT)],
                        out_hbm.at[cid].at[pl.ds(base, RPT)])

    f = pl.kernel(body, out_type=out_type, mesh=mesh, scratch_types=scratch)
    return f(hw, src3, dst3)


def _sc_degree(dst3):
    mesh = plsc.VectorSubcoreMesh(core_axis_name="c", subcore_axis_name="s",
                                  num_cores=NC, num_subcores=NS)

    def body(dst_hbm, zo_hbm, deg_hbm, dst_v, ones_v, dacc):
        cid = lax.axis_index("c")
        sid = lax.axis_index("s")
        wid = cid * NS + sid

        pltpu.sync_copy(dst_hbm.at[wid], dst_v)
        pltpu.sync_copy(zo_hbm.at[0], ones_v)
        base = sid * RPT
        for k in range(RPT // CHUNK):
            pltpu.sync_copy(ones_v, dacc.at[pl.ds(base + k * CHUNK, CHUNK)])
        pltpu.sync_copy(zo_hbm.at[1], ones_v)

        plsc.subcore_barrier()

        def step(j, carry):
            pltpu.sync_copy(ones_v, dacc.at[dst_v.at[j]], add=True)
            return carry
        lax.fori_loop(0, CPT, step, 0)

        plsc.subcore_barrier()

        pltpu.sync_copy(dacc.at[pl.ds(base, RPT)],
                        deg_hbm.at[cid].at[pl.ds(base, RPT)])

    f = pl.kernel(
        body,
        out_type=jax.ShapeDtypeStruct((NC, N_PAD, 8), jnp.float32),
        mesh=mesh,
        scratch_types=[
            pltpu.VMEM((CPT, CHUNK), jnp.int32),
            pltpu.VMEM((CHUNK, 8), jnp.float32),
            pltpu.VMEM_SHARED((N_PAD, 8), jnp.float32),
        ],
        compiler_params=pltpu.CompilerParams(use_tc_tiling_on_sc=False),
    )
    zo = jnp.concatenate([jnp.zeros((1, CHUNK, 8), jnp.float32),
                          jnp.ones((1, CHUNK, 8), jnp.float32)])
    return f(dst3, zo)


def _comb1_body(agg_ref, degp_ref, w_ref, h1_ref, hw2_ref, deg_ref):
    agg = agg_ref[0] + agg_ref[1]
    deg = jnp.maximum(degp_ref[0, :, :1] + degp_ref[1, :, :1], 1.0)
    h1 = jnp.maximum(agg, 0.0) / deg
    h1_ref[...] = h1
    hw2_ref[...] = jnp.dot(h1, w_ref[...], preferred_element_type=jnp.float32)
    deg_ref[...] = deg


def _tc_combine1(agg_parts, deg_parts, w_enc2):
    return pl.pallas_call(
        _comb1_body,
        grid=(N_PAD // BM,),
        in_specs=[
            pl.BlockSpec((NC, BM, D), lambda i: (0, i, 0)),
            pl.BlockSpec((NC, BM, 8), lambda i: (0, i, 0)),
            pl.BlockSpec((D, D), lambda i: (0, 0)),
        ],
        out_specs=[
            pl.BlockSpec((BM, D), lambda i: (i, 0)),
            pl.BlockSpec((BM, D), lambda i: (i, 0)),
            pl.BlockSpec((BM, 1), lambda i: (i, 0)),
        ],
        out_shape=[
            jax.ShapeDtypeStruct((N_PAD, D), jnp.float32),
            jax.ShapeDtypeStruct((N_PAD, D), jnp.float32),
            jax.ShapeDtypeStruct((N_PAD, 1), jnp.float32),
        ],
    )(agg_parts, deg_parts, w_enc2)


def _dec_body(agg_ref, deg_ref, h1_ref, wda_ref, wdb_ref, wm_ref,
              wc_ref, wv_ref, ws_ref, logits_ref, c_ref, v_ref, f_ref):
    agg = agg_ref[0] + agg_ref[1]
    deg = deg_ref[...]
    h2 = jnp.maximum(agg, 0.0) / deg
    hd = jnp.dot(h2, wda_ref[...], preferred_element_type=jnp.float32)
    hd = hd + jnp.dot(h1_ref[...], wdb_ref[...],
                      preferred_element_type=jnp.float32)
    hd = jnp.maximum(hd, 0.0)
    y = jnp.dot(hd, wm_ref[...], preferred_element_type=jnp.float32)
    f = jnp.where(y >= 0.0, y, 0.1 * y)
    f_ref[...] = f
    cz = jnp.dot(f, wc_ref[...], preferred_element_type=jnp.float32)
    c_ref[...] = 1.0 / (1.0 + jnp.exp(-cz))
    vz = jnp.dot(f, wv_ref[...], preferred_element_type=jnp.float32)
    v_ref[...] = jnp.maximum(vz, 0.0)
    logits_ref[...] = jnp.dot(f, ws_ref[...],
                              preferred_element_type=jnp.float32)


def _tc_decode(agg2_parts, deg, h1, w_dec, w_mlp, w_center, w_var, w_softmax):
    wda = w_dec[:D]
    wdb = w_dec[D:]
    nc1 = w_center.shape[1]
    nv = w_var.shape[1]
    ns = w_softmax.shape[1]
    return pl.pallas_call(
        _dec_body,
        grid=(N_PAD // BM,),
        in_specs=[
            pl.BlockSpec((NC, BM, D), lambda i: (0, i, 0)),
            pl.BlockSpec((BM, 1), lambda i: (i, 0)),
            pl.BlockSpec((BM, D), lambda i: (i, 0)),
            pl.BlockSpec((D, D), lambda i: (0, 0)),
            pl.BlockSpec((D, D), lambda i: (0, 0)),
            pl.BlockSpec((D, D), lambda i: (0, 0)),
            pl.BlockSpec((D, nc1), lambda i: (0, 0)),
            pl.BlockSpec((D, nv), lambda i: (0, 0)),
            pl.BlockSpec((D, ns), lambda i: (0, 0)),
        ],
        out_specs=[
            pl.BlockSpec((BM, ns), lambda i: (i, 0)),
            pl.BlockSpec((BM, nc1), lambda i: (i, 0)),
            pl.BlockSpec((BM, nv), lambda i: (i, 0)),
            pl.BlockSpec((BM, D), lambda i: (i, 0)),
        ],
        out_shape=[
            jax.ShapeDtypeStruct((N_PAD, ns), jnp.float32),
            jax.ShapeDtypeStruct((N_PAD, nc1), jnp.float32),
            jax.ShapeDtypeStruct((N_PAD, nv), jnp.float32),
            jax.ShapeDtypeStruct((N_PAD, D), jnp.float32),
        ],
    )(agg2_parts, deg, h1, wda, wdb, w_mlp, w_center, w_var, w_softmax)


def kernel(x, edge_index, W_enc1, W_enc2, W_dec, W_mlp, W_center, W_var,
           W_softmax):
    src = edge_index[0]
    dst = edge_index[1]
    pad = E_PAD - E
    pad_src = jnp.arange(pad, dtype=jnp.int32) % N
    pad_dst = N + jnp.arange(pad, dtype=jnp.int32) % (N_PAD - N)
    src3 = jnp.concatenate([src, pad_src]).reshape(NW, CPT, CHUNK)
    dst3 = jnp.concatenate([dst, pad_dst]).reshape(NW, CPT, CHUNK)

    x_p = jnp.concatenate([x, jnp.zeros((N_PAD - N, D), jnp.float32)])
    hw1 = _tc_matmul(x_p, W_enc1)
    deg_parts = _sc_degree(dst3)
    agg1_parts = _sc_msgpass(hw1, src3, dst3)
    h1, hw2, deg = _tc_combine1(agg1_parts, deg_parts, W_enc2)
    agg2_parts = _sc_msgpass(hw2, src3, dst3)
    logits, c, v, f = _tc_decode(agg2_parts, deg, h1, W_dec, W_mlp,
                                 W_center, W_var, W_softmax)
    return (logits[:N], c[:N], v[:N], f[:N])
